# baseline (speedup 1.0000x reference)
import jax
import jax.numpy as jnp
from jax import lax
from jax.experimental import pallas as pl
from jax.experimental.pallas import tpu as pltpu
from jax.experimental.pallas import tpu_sc as plsc

B = 1024
S = 1001
E = 64
NC = 2
NS = 16
NW = NC * NS
S_PER_W = 32
NBUF = 4


def _emb_body(ut_hbm, dt_hbm, post_hbm, tabt_hbm, out,
              tabt_v, pos_sv, pos_col, idx_v, dsv, b0, b1, b2, b3,
              ss0, ss1, ss2, ss3, stg):
    wid = lax.axis_index("s") * NC + lax.axis_index("c")
    s0 = wid * S_PER_W
    scount = jnp.maximum(0, jnp.minimum(S_PER_W, S - s0))
    bufs = [b0, b1, b2, b3]
    ssems = [ss0, ss1, ss2, ss3]

    pltpu.sync_copy(tabt_hbm, tabt_v)
    pltpu.sync_copy(post_hbm.at[:, pl.ds(s0, S_PER_W)], pos_sv)

    def drain_store(p, s):
        pltpu.make_async_copy(out.at[s].at[p % (E // 8)],
                              bufs[p], ssems[p]).wait()

    ri = lax.iota(jnp.int32, 16)

    def s_body(s_loc, carry):
        s = s0 + s_loc
        c1 = pltpu.async_copy(ut_hbm.at[s], idx_v, stg)
        c2 = pltpu.async_copy(dt_hbm.at[s], dsv, stg)
        c1.wait()
        c2.wait()
        zvec = idx_v[pl.ds(0, 16)] >> 31
        scol = zvec + s_loc
        for t in range(E // 16):
            pos_col[pl.ds(t * 16, 16)] = plsc.load_gather(
                pos_sv, [t * 16 + ri, scol])

        def etg_body(et_g, c):
            for k in range(NBUF):
                p = k
                slab = NBUF * et_g + k
                bufp = bufs[p]

                @pl.when((s_loc > 0) | (et_g > 0))
                def _():
                    drain_store(p, s)

                esps = [zvec + slab * 8 + ei for ei in range(8)]
                psps = [plsc.load_gather(pos_col, [esps[ei]])
                        for ei in range(8)]

                def blk_body(blk, c2, _bufp=bufp, _esps=esps, _psps=psps):
                    ivecs = [idx_v[pl.ds(blk * 128 + j * 16, 16)]
                             for j in range(8)]
                    for ei in range(8):
                        for j in range(8):
                            g = plsc.load_gather(tabt_v, [_esps[ei], ivecs[j]])
                            _bufp[blk, ei, pl.ds(j * 16, 16)] = g + _psps[ei]
                    return c2

                lax.fori_loop(0, 8, blk_body, 0)

                if k == NBUF - 1:
                    @pl.when(et_g == (E // 8) // NBUF - 1)
                    def _(_bufp=bufp):
                        def dadd(blk, c3):
                            for j in range(8):
                                sl = pl.ds(j * 16, 16)
                                dv = dsv[pl.ds(blk * 128 + j * 16, 16)]
                                _bufp[blk, 7, sl] = _bufp[blk, 7, sl] + dv
                            return c3
                        lax.fori_loop(0, 8, dadd, 0)

                pltpu.async_copy(bufp, out.at[s].at[slab], ssems[p])
            return c

        lax.fori_loop(0, (E // 8) // NBUF, etg_body, 0)
        return carry

    lax.fori_loop(0, scount, s_body, 0)

    last = s0 + scount - 1

    @pl.when(scount > 0)
    def _():
        for p in range(NBUF):
            drain_store(p, last)


def kernel(used_symbols, distribution, pos_encoding, symbol_embeddings):
    ut = used_symbols.astype(jnp.int32).T
    dt = distribution.T
    post = jnp.pad(pos_encoding.T, ((0, 0), (0, NW * S_PER_W - S)))
    tabt = jnp.pad(symbol_embeddings, ((0, 0), (0, 1))).T

    mesh = plsc.VectorSubcoreMesh(core_axis_name="c", subcore_axis_name="s")
    run = pl.kernel(
        _emb_body,
        out_type=jax.ShapeDtypeStruct((S, E // 8, B // 128, 8, 128),
                                      jnp.float32),
        mesh=mesh,
        scratch_types=[
            pltpu.VMEM((E, S), jnp.float32),
            pltpu.VMEM((E, S_PER_W), jnp.float32),
            pltpu.VMEM((E,), jnp.float32),
            pltpu.VMEM((B,), jnp.int32),
            pltpu.VMEM((B,), jnp.float32),
            pltpu.VMEM((B // 128, 8, 128), jnp.float32),
            pltpu.VMEM((B // 128, 8, 128), jnp.float32),
            pltpu.VMEM((B // 128, 8, 128), jnp.float32),
            pltpu.VMEM((B // 128, 8, 128), jnp.float32),
            pltpu.SemaphoreType.DMA,
            pltpu.SemaphoreType.DMA,
            pltpu.SemaphoreType.DMA,
            pltpu.SemaphoreType.DMA,
            pltpu.SemaphoreType.DMA,
        ],
        compiler_params=pltpu.CompilerParams(use_tc_tiling_on_sc=False,
                                             needs_layout_passes=False),
    )
    out5 = run(ut, dt, post, tabt)
    return out5.transpose(2, 4, 0, 1, 3).reshape(B, S, E)


# speedup vs baseline: 1.1774x; 1.1774x over previous
import jax
import jax.numpy as jnp
from jax import lax
from jax.experimental import pallas as pl
from jax.experimental.pallas import tpu as pltpu
from jax.experimental.pallas import tpu_sc as plsc

B = 1024
S = 1001
E = 64
NC = 2
NS = 16
NW = NC * NS
S_PER_W = 32
NBUF = 4


def _emb_body(ut_hbm, dt_hbm, post_hbm, tabt_hbm, out,
              tabt_v, pos_sv, pos_col, idx_v, dsv, b0, b1, b2, b3,
              ss0, ss1, ss2, ss3, stg):
    wid = lax.axis_index("s") * NC + lax.axis_index("c")
    s0 = wid * S_PER_W
    scount = jnp.maximum(0, jnp.minimum(S_PER_W, S - s0))
    bufs = [b0, b1, b2, b3]
    ssems = [ss0, ss1, ss2, ss3]

    pltpu.sync_copy(tabt_hbm, tabt_v)
    pltpu.sync_copy(post_hbm.at[:, pl.ds(s0, S_PER_W)], pos_sv)

    def drain_store(p, s):
        pltpu.make_async_copy(out.at[s].at[p % (E // 8)],
                              bufs[p], ssems[p]).wait()

    ri = lax.iota(jnp.int32, 16)

    def s_body(s_loc, carry):
        s = s0 + s_loc
        c1 = pltpu.async_copy(ut_hbm.at[s], idx_v, stg)
        c2 = pltpu.async_copy(dt_hbm.at[s], dsv, stg)
        c1.wait()
        c2.wait()
        zvec = idx_v[pl.ds(0, 16)] >> 31
        scol = zvec + s_loc
        for t in range(E // 16):
            pos_col[pl.ds(t * 16, 16)] = plsc.load_gather(
                pos_sv, [t * 16 + ri, scol])

        for et in range(E // 8):
            p = et % NBUF
            if et < NBUF:
                @pl.when(s_loc > 0)
                def _():
                    drain_store(p, s)
            else:
                drain_store(p, s)
            bufp = bufs[p]

            def blk_body(blk, c, _et=et, _bufp=bufp):
                ivecs = [idx_v[pl.ds(blk * 128 + j * 16, 16)]
                         for j in range(8)]

                def e_body(e_i, c2):
                    esp = ri * 0 + (jnp.int32(_et * 8) + e_i)
                    psp = plsc.load_gather(pos_col, [esp])
                    for j in range(8):
                        g = plsc.load_gather(tabt_v, [esp, ivecs[j]])
                        _bufp[blk, e_i, pl.ds(j * 16, 16)] = g + psp
                    return c2

                lax.fori_loop(0, 8, e_body, 0)

                if _et == E // 8 - 1:
                    for j in range(8):
                        sl = pl.ds(j * 16, 16)
                        dv = dsv[pl.ds(blk * 128 + j * 16, 16)]
                        _bufp[blk, 7, sl] = _bufp[blk, 7, sl] + dv
                return c

            lax.fori_loop(0, 8, blk_body, 0)

            pltpu.async_copy(bufp, out.at[s].at[et], ssems[p])
        return carry

    lax.fori_loop(0, scount, s_body, 0)

    last = s0 + scount - 1

    @pl.when(scount > 0)
    def _():
        for p in range(NBUF):
            drain_store(p, last)


def kernel(used_symbols, distribution, pos_encoding, symbol_embeddings):
    ut = used_symbols.astype(jnp.int32).T
    dt = distribution.T
    post = jnp.pad(pos_encoding.T, ((0, 0), (0, NW * S_PER_W - S)))
    tabt = jnp.pad(symbol_embeddings, ((0, 0), (0, 1))).T

    mesh = plsc.VectorSubcoreMesh(core_axis_name="c", subcore_axis_name="s")
    run = pl.kernel(
        _emb_body,
        out_type=jax.ShapeDtypeStruct((S, E // 8, B // 128, 8, 128),
                                      jnp.float32),
        mesh=mesh,
        scratch_types=[
            pltpu.VMEM((E, S), jnp.float32),
            pltpu.VMEM((E, S_PER_W), jnp.float32),
            pltpu.VMEM((E,), jnp.float32),
            pltpu.VMEM((B,), jnp.int32),
            pltpu.VMEM((B,), jnp.float32),
            pltpu.VMEM((B // 128, 8, 128), jnp.float32),
            pltpu.VMEM((B // 128, 8, 128), jnp.float32),
            pltpu.VMEM((B // 128, 8, 128), jnp.float32),
            pltpu.VMEM((B // 128, 8, 128), jnp.float32),
            pltpu.SemaphoreType.DMA,
            pltpu.SemaphoreType.DMA,
            pltpu.SemaphoreType.DMA,
            pltpu.SemaphoreType.DMA,
            pltpu.SemaphoreType.DMA,
        ],
        compiler_params=pltpu.CompilerParams(use_tc_tiling_on_sc=False,
                                             needs_layout_passes=False),
    )
    out5 = run(ut, dt, post, tabt)
    return out5.transpose(2, 4, 0, 1, 3).reshape(B, S, E)
